# Initial kernel scaffold; baseline (speedup 1.0000x reference)
import jax
import jax.numpy as jnp
from jax.experimental import pallas as pl


def kernel(x, edge_index, edge_weight, W1b_w, W1b_b, W1a_w, W1a_b, W2b_w, W2b_b, W2a_w, W2a_b, bn_gamma, bn_beta, bn_mean, bn_var, fc1_w, fc1_b, fc2_w, fc2_b):
    raise NotImplementedError("write your pallas kernel here")


# speedup vs baseline: 26.4413x; 26.4413x over previous
import functools

import jax
import jax.numpy as jnp
from jax import lax
from jax.experimental import pallas as pl
from jax.experimental.pallas import tpu as pltpu
from jax.experimental.pallas import tpu_sc as plsc

N = 10000
E = 160000
D_IN = 128
LAM = 1.0
ALP = 1.0 / (1.0 + LAM)
PROP = 64

NPAD = 10240
NT = 16
ROWS_PT = NPAD // NT
EDG_PT = E // NT
CH = 80
NCH = EDG_PT // CH
DH = 16

_mesh = plsc.VectorSubcoreMesh(core_axis_name="c", subcore_axis_name="s")


@functools.partial(
    pl.kernel,
    out_type=jax.ShapeDtypeStruct((NPAD, DH), jnp.float32),
    mesh=_mesh,
    compiler_params=pltpu.CompilerParams(use_tc_tiling_on_sc=False),
    scratch_types=[
        pltpu.VMEM_SHARED((NPAD, DH), jnp.float32),
        pltpu.VMEM((NCH, CH), jnp.int32),
        pltpu.VMEM((CH, DH), jnp.float32),
        pltpu.VMEM((ROWS_PT, DH), jnp.float32),
    ],
)
def _deg_kernel(dst_hbm, out_hbm, deg_sp, idx_v, ones_v, zstage_v):
    c = lax.axis_index("c")
    s = lax.axis_index("s")

    @pl.when(c == 0)
    def _():
        r0 = s * ROWS_PT

        def fill_z(r, _):
            zstage_v[r, :] = jnp.zeros((DH,), jnp.float32)
            return 0

        lax.fori_loop(0, ROWS_PT, fill_z, 0)

        def fill_o(r, _):
            ones_v[r, :] = jnp.ones((DH,), jnp.float32)
            return 0

        lax.fori_loop(0, CH, fill_o, 0)

        pltpu.sync_copy(dst_hbm.at[s], idx_v)
        pltpu.sync_copy(zstage_v, deg_sp.at[pl.ds(r0, ROWS_PT)])
        plsc.subcore_barrier()

        def body(j, _):
            pltpu.sync_copy(ones_v, deg_sp.at[idx_v.at[j]], add=True)
            return 0

        lax.fori_loop(0, NCH, body, 0)
        plsc.subcore_barrier()

        pltpu.sync_copy(deg_sp.at[pl.ds(r0, ROWS_PT)], zstage_v)
        pltpu.sync_copy(zstage_v, out_hbm.at[pl.ds(r0, ROWS_PT)])


@functools.partial(
    pl.kernel,
    out_type=jax.ShapeDtypeStruct((2, NPAD, DH), jnp.float32),
    mesh=_mesh,
    compiler_params=pltpu.CompilerParams(use_tc_tiling_on_sc=False),
    scratch_types=[
        pltpu.VMEM_SHARED((NPAD, DH), jnp.float32),
        pltpu.VMEM_SHARED((NPAD, DH), jnp.float32),
        pltpu.VMEM((NCH, CH), jnp.int32),
        pltpu.VMEM((NCH, CH), jnp.int32),
        pltpu.VMEM((2, CH, DH), jnp.float32),
        pltpu.VMEM((ROWS_PT, DH), jnp.float32),
        pltpu.VMEM((ROWS_PT, DH), jnp.float32),
        pltpu.VMEM((ROWS_PT, DH), jnp.float32),
        pltpu.VMEM((ROWS_PT, DH), jnp.float32),
        pltpu.VMEM((ROWS_PT, DH), jnp.float32),
        pltpu.SemaphoreType.DMA,
    ],
)
def _prop_kernel(u0_hbm, b_hbm, c_hbm, src_hbm, dst_hbm, out_hbm,
                 u_sp, au_sp, src_v, dst_v, stage_v,
                 uloc, bloc, cloc, auloc, zbuf, gsem):
    c = lax.axis_index("c")
    s = lax.axis_index("s")
    r0 = s * ROWS_PT
    rows = pl.ds(r0, ROWS_PT)

    pltpu.sync_copy(src_hbm.at[s], src_v)
    pltpu.sync_copy(dst_hbm.at[s], dst_v)
    pltpu.sync_copy(u0_hbm.at[c, rows], uloc)
    pltpu.sync_copy(b_hbm.at[c, rows], bloc)
    pltpu.sync_copy(c_hbm.at[rows], cloc)
    pltpu.sync_copy(uloc, u_sp.at[rows])

    def fill_z(r, _):
        zbuf[r, :] = jnp.zeros((DH,), jnp.float32)
        return 0

    lax.fori_loop(0, ROWS_PT, fill_z, 0)
    pltpu.sync_copy(zbuf, au_sp.at[rows])
    plsc.subcore_barrier()

    def step(_, carry):
        pltpu.make_async_copy(u_sp.at[src_v.at[0]], stage_v.at[0], gsem).start()

        def chunk(j, _):
            jb = lax.rem(j, 2)
            pltpu.make_async_copy(
                u_sp.at[src_v.at[j]], stage_v.at[jb], gsem).wait()

            @pl.when(j < NCH - 1)
            def _():
                pltpu.make_async_copy(
                    u_sp.at[src_v.at[j + 1]],
                    stage_v.at[lax.rem(j + 1, 2)], gsem).start()

            pltpu.sync_copy(stage_v.at[jb], au_sp.at[dst_v.at[j]], add=True)
            return 0

        lax.fori_loop(0, NCH, chunk, 0)
        plsc.subcore_barrier()

        pltpu.sync_copy(au_sp.at[rows], auloc)
        pltpu.sync_copy(zbuf, au_sp.at[rows])

        def upd(r, _):
            u = uloc[r, :]
            uloc[r, :] = 0.5 * u + cloc[r, :] * auloc[r, :] + bloc[r, :]
            return 0

        lax.fori_loop(0, ROWS_PT, upd, 0)
        pltpu.sync_copy(uloc, u_sp.at[rows])
        plsc.subcore_barrier()
        return carry

    lax.fori_loop(0, PROP, step, 0)

    pltpu.sync_copy(uloc, out_hbm.at[c, rows])


def _leaky(v):
    return jnp.where(v > 0, v, 0.01 * v)


def _prep1_body(x_ref, w1b_ref, b1b_ref, w1a_ref, degb_ref,
                u0_ref, bm_ref, c16_ref):
    deg = degb_ref[:, 0:1]
    dbias = LAM * deg + 1.0
    s = lax.rsqrt(dbias)
    inv1 = 1.0 / dbias
    w1 = jnp.dot(w1a_ref[...], w1b_ref[...],
                 preferred_element_type=jnp.float32)
    c1 = jnp.dot(w1a_ref[...], b1b_ref[0, :],
                 preferred_element_type=jnp.float32)
    z0 = jnp.dot(x_ref[...], w1.T,
                 preferred_element_type=jnp.float32) + c1
    u0 = s * z0
    bm = ALP * (s * inv1) * z0
    u0_ref[0, :, :] = u0[:, :DH]
    u0_ref[1, :, :] = u0[:, DH:]
    bm_ref[0, :, :] = bm[:, :DH]
    bm_ref[1, :, :] = bm[:, DH:]
    c16_ref[...] = jnp.broadcast_to(ALP * LAM * (s * s), (NPAD, DH))


def _prep2_body(u1_ref, degb_ref, b1a_ref, w2b_ref, b2b_ref, w2a_ref,
                u0_ref, bm_ref):
    deg = degb_ref[:, 0:1]
    dbias = LAM * deg + 1.0
    s = lax.rsqrt(dbias)
    inv1 = 1.0 / dbias
    sinv = jnp.sqrt(dbias)
    z1 = jnp.concatenate([u1_ref[0, :, :], u1_ref[1, :, :]], axis=1) * sinv
    h1 = _leaky(z1 + b1a_ref[0, :])
    w2 = jnp.dot(w2a_ref[...], w2b_ref[...],
                 preferred_element_type=jnp.float32)
    c2 = jnp.dot(w2a_ref[...], b2b_ref[0, :],
                 preferred_element_type=jnp.float32)
    z02 = jnp.dot(h1, w2.T, preferred_element_type=jnp.float32) + c2
    z02 = jnp.concatenate(
        [z02, jnp.zeros((NPAD, 2 * DH - 20), jnp.float32)], axis=1)
    u0 = s * z02
    bm = ALP * (s * inv1) * z02
    u0_ref[0, :, :] = u0[:, :DH]
    u0_ref[1, :, :] = u0[:, DH:]
    bm_ref[0, :, :] = bm[:, :DH]
    bm_ref[1, :, :] = bm[:, DH:]


def _head_body(u2_ref, degb_ref, b2a_ref, g_ref, b_ref, m_ref, v_ref,
               fc1w_ref, fc1b_ref, fc2w_ref, fc2b_ref, out_ref):
    deg = degb_ref[:, 0:1]
    dbias = LAM * deg + 1.0
    sinv = jnp.sqrt(dbias)
    z2 = jnp.concatenate([u2_ref[0, :, :], u2_ref[1, :, :]], axis=1) * sinv
    z2 = z2[:N, :]
    h = z2[:, :20] + b2a_ref[0, :]
    h = (h - m_ref[0, :]) / jnp.sqrt(v_ref[0, :] + 1e-5) * g_ref[0, :] \
        + b_ref[0, :]
    h = _leaky(h)
    pooled = jnp.sum(h, axis=0, keepdims=True)
    o = _leaky(jnp.dot(pooled, fc1w_ref[...].T,
                       preferred_element_type=jnp.float32) + fc1b_ref[0, :])
    out_ref[...] = jnp.dot(o, fc2w_ref[...].T,
                           preferred_element_type=jnp.float32) + fc2b_ref[0, :]


def kernel(x, edge_index, edge_weight, W1b_w, W1b_b, W1a_w, W1a_b,
           W2b_w, W2b_b, W2a_w, W2a_b, bn_gamma, bn_beta, bn_mean, bn_var,
           fc1_w, fc1_b, fc2_w, fc2_b):
    src3 = edge_index[0].reshape(NT, NCH, CH)
    dst3 = edge_index[1].reshape(NT, NCH, CH)
    xp = jnp.pad(x, ((0, NPAD - N), (0, 0)))

    degb = _deg_kernel(dst3)

    u0, bm, c16 = pl.pallas_call(
        _prep1_body,
        out_shape=[
            jax.ShapeDtypeStruct((2, NPAD, DH), jnp.float32),
            jax.ShapeDtypeStruct((2, NPAD, DH), jnp.float32),
            jax.ShapeDtypeStruct((NPAD, DH), jnp.float32),
        ],
    )(xp, W1b_w, W1b_b.reshape(1, -1), W1a_w, degb)

    u1 = _prop_kernel(u0, bm, c16, src3, dst3)

    u02, bm2 = pl.pallas_call(
        _prep2_body,
        out_shape=[
            jax.ShapeDtypeStruct((2, NPAD, DH), jnp.float32),
            jax.ShapeDtypeStruct((2, NPAD, DH), jnp.float32),
        ],
    )(u1, degb, W1a_b.reshape(1, -1), W2b_w, W2b_b.reshape(1, -1), W2a_w)

    u2 = _prop_kernel(u02, bm2, c16, src3, dst3)

    out = pl.pallas_call(
        _head_body,
        out_shape=jax.ShapeDtypeStruct((1, 2), jnp.float32),
    )(u2, degb, W2a_b.reshape(1, -1), bn_gamma.reshape(1, -1),
      bn_beta.reshape(1, -1), bn_mean.reshape(1, -1), bn_var.reshape(1, -1),
      fc1_w, fc1_b.reshape(1, -1), fc2_w, fc2_b.reshape(1, -1))

    return out
